# Initial kernel scaffold; baseline (speedup 1.0000x reference)
import jax
import jax.numpy as jnp
from jax.experimental import pallas as pl


def kernel(x, embed):
    raise NotImplementedError("write your pallas kernel here")


# speedup vs baseline: 1.8247x; 1.8247x over previous
import functools

import jax
import jax.numpy as jnp
from jax import lax
from jax.experimental import pallas as pl
from jax.experimental.pallas import tpu as pltpu
from jax.experimental.pallas import tpu_sc as plsc

N_TOKENS = 16384
K_CODES = 8192
DIM = 32

TOK_TILE = 256
GRID = N_TOKENS // TOK_TILE

_NC, _NS = 2, 16
_NW = _NC * _NS
_IDX_CHUNK = 128
_ROWS_PER_W = N_TOKENS // _NW // _IDX_CHUNK
_N_IDX_ROWS = N_TOKENS // _IDX_CHUNK


def _argmax_body(xt_ref, embed_ref, esq_ref, idx_ref):
    xbt = xt_ref[...].astype(jnp.bfloat16)
    e = embed_ref[...]
    dott = lax.dot_general(e, xbt, (((1,), (0,)), ((), ())),
                           preferred_element_type=jnp.float32)
    dist = 2.0 * dott - esq_ref[...].reshape(K_CODES, 1)
    idx = jnp.argmax(dist, axis=0).astype(jnp.int32)
    idx_ref[0, 0, :] = idx


_argmax_call = pl.pallas_call(
    _argmax_body,
    grid=(GRID,),
    in_specs=[
        pl.BlockSpec((DIM, TOK_TILE), lambda i: (0, i)),
        pl.BlockSpec((K_CODES, DIM), lambda i: (0, 0)),
        pl.BlockSpec((1, K_CODES), lambda i: (0, 0)),
    ],
    out_specs=pl.BlockSpec((1, 1, TOK_TILE), lambda i: (i, 0, 0)),
    out_shape=jax.ShapeDtypeStruct((GRID, 1, TOK_TILE), jnp.int32),
)


def _gather_body(embed_hbm, idx_hbm, out_hbm, idx_v, rows_v, sem):
    wid = lax.axis_index("s") * _NC + lax.axis_index("c")
    base = wid * _ROWS_PER_W
    pltpu.sync_copy(idx_hbm.at[pl.ds(base, _ROWS_PER_W)], idx_v)
    copies = [
        pltpu.async_copy(embed_hbm.at[idx_v.at[j]], rows_v.at[j], sem)
        for j in range(_ROWS_PER_W)
    ]
    for c in copies:
        c.wait()
    pltpu.sync_copy(rows_v, out_hbm.at[pl.ds(base, _ROWS_PER_W)])


@functools.cache
def _gather_call():
    return pl.kernel(
        _gather_body,
        out_type=jax.ShapeDtypeStruct((_N_IDX_ROWS, _IDX_CHUNK, DIM), jnp.float32),
        mesh=plsc.VectorSubcoreMesh(core_axis_name="c", subcore_axis_name="s"),
        scratch_types=[
            pltpu.VMEM((_ROWS_PER_W, _IDX_CHUNK), jnp.int32),
            pltpu.VMEM((_ROWS_PER_W, _IDX_CHUNK, DIM), jnp.float32),
            pltpu.SemaphoreType.DMA,
        ],
        compiler_params=pltpu.CompilerParams(use_tc_tiling_on_sc=False),
    )


def kernel(x, embed):
    shape = x.shape
    x_flat = x.reshape(-1, DIM)
    esq = jnp.sum(embed.T ** 2, axis=0, keepdims=True)
    idx = _argmax_call(x_flat.T, embed, esq).reshape(N_TOKENS)
    quant = _gather_call()(embed, idx.reshape(_N_IDX_ROWS, _IDX_CHUNK))
    return (quant.reshape(shape), idx.reshape(shape[:-1]))
